# initial kernel scaffold (unmeasured)
import jax
import jax.numpy as jnp
from jax import lax
from jax.experimental import pallas as pl
from jax.experimental.pallas import tpu as pltpu

S = 1024
D = 2048
H = 16
DH = 128
DR = 32
SCALE = (DH + DR) ** -0.5
BF16 = jnp.bfloat16
F32 = jnp.float32


def kernel(x, Wdkv, Wuk, Wuv, Wq, Wqr, Wkr, Wo):
    xb = x.reshape(S, D).astype(BF16)
    wdkv = Wdkv.astype(BF16)
    wuk = Wuk.astype(BF16)
    wuv = Wuv.astype(BF16)
    wq = Wq.astype(BF16)
    wqr = Wqr.astype(BF16)
    wkr = Wkr.astype(BF16)
    wo = Wo.astype(BF16)

    def body(x_ref, wdkv_ref, wuk_ref, wuv_ref, wq_ref, wqr_ref, wkr_ref,
             wo_ref, out_ref, kp_ref, vp_ref, krecv_ref, vrecv_ref,
             send_sems, recv_sems):
        my_x = lax.axis_index("x")
        my_y = lax.axis_index("y")
        my_z = lax.axis_index("z")
        peer = (my_x, 1 - my_y, my_z)

        barrier = pltpu.get_barrier_semaphore()
        pl.semaphore_signal(barrier, inc=1, device_id=peer,
                            device_id_type=pl.DeviceIdType.MESH)
        pl.semaphore_wait(barrier, 1)

        xv = x_ref[...]
        c = jnp.dot(xv, wdkv_ref[...], preferred_element_type=F32)
        c = c.astype(BF16)
        kp_ref[...] = jnp.dot(c, wuk_ref[...],
                              preferred_element_type=F32).astype(BF16)
        vp_ref[...] = jnp.dot(c, wuv_ref[...],
                              preferred_element_type=F32).astype(BF16)

        rdma_k = pltpu.make_async_remote_copy(
            src_ref=kp_ref, dst_ref=krecv_ref,
            send_sem=send_sems.at[0], recv_sem=recv_sems.at[0],
            device_id=peer, device_id_type=pl.DeviceIdType.MESH)
        rdma_k.start()
        rdma_v = pltpu.make_async_remote_copy(
            src_ref=vp_ref, dst_ref=vrecv_ref,
            send_sem=send_sems.at[1], recv_sem=recv_sems.at[1],
            device_id=peer, device_id_type=pl.DeviceIdType.MESH)
        rdma_v.start()

        q = jnp.dot(xv, wq_ref[...], preferred_element_type=F32).astype(BF16)
        qr = jnp.dot(xv, wqr_ref[...], preferred_element_type=F32).astype(BF16)
        kr = jnp.dot(xv, wkr_ref[...], preferred_element_type=F32).astype(BF16)

        rdma_k.wait()
        rdma_v.wait()

        k = (kp_ref[...].astype(F32) + krecv_ref[...].astype(F32)).astype(BF16)
        v = (vp_ref[...].astype(F32) + vrecv_ref[...].astype(F32)).astype(BF16)

        outs = []
        for h in range(H):
            q_h = q[:, h * DH:(h + 1) * DH]
            k_h = k[:, h * DH:(h + 1) * DH]
            v_h = v[:, h * DH:(h + 1) * DH]
            qr_h = qr[:, h * DR:(h + 1) * DR]
            s1 = lax.dot_general(q_h, k_h, (((1,), (1,)), ((), ())),
                                 preferred_element_type=F32)
            s2 = lax.dot_general(qr_h, kr, (((1,), (1,)), ((), ())),
                                 preferred_element_type=F32)
            sc = (s1 + s2) * SCALE
            m = jnp.max(sc, axis=1, keepdims=True)
            p = jnp.exp(sc - m)
            p = (p / jnp.sum(p, axis=1, keepdims=True)).astype(BF16)
            outs.append(jnp.dot(p, v_h,
                                preferred_element_type=F32).astype(BF16))
        o = jnp.concatenate(outs, axis=1)
        out_ref[...] = jnp.dot(o, wo_ref[...], preferred_element_type=F32)

    out = pl.pallas_call(
        body,
        out_shape=jax.ShapeDtypeStruct((S, D), F32),
        in_specs=[pl.BlockSpec(memory_space=pltpu.VMEM)] * 8,
        out_specs=pl.BlockSpec(memory_space=pltpu.VMEM),
        scratch_shapes=[
            pltpu.VMEM((S, D), BF16),
            pltpu.VMEM((S, D), BF16),
            pltpu.VMEM((S, D), BF16),
            pltpu.VMEM((S, D), BF16),
            pltpu.SemaphoreType.DMA((2,)),
            pltpu.SemaphoreType.DMA((2,)),
        ],
        compiler_params=pltpu.CompilerParams(collective_id=0),
    )(xb, wdkv, wuk, wuv, wq, wqr, wkr, wo)
    return out.reshape(1, S, D)


# baseline (device time: 199685 ns/iter reference)
import jax
import jax.numpy as jnp
from jax import lax
from jax.experimental import pallas as pl
from jax.experimental.pallas import tpu as pltpu

S = 1024
D = 2048
H = 16
DH = 128
DR = 32
SCALE = (DH + DR) ** -0.5
BF16 = jnp.bfloat16
F32 = jnp.float32


def kernel(x, Wdkv, Wuk, Wuv, Wq, Wqr, Wkr, Wo):
    xb = x.reshape(S, D).astype(BF16)
    wdkv = Wdkv.astype(BF16)
    wuk = Wuk.astype(BF16)
    wuv = Wuv.astype(BF16)
    wq = Wq.astype(BF16)
    wqr = Wqr.astype(BF16)
    wkr = Wkr.astype(BF16)
    wo = Wo.astype(BF16)

    def body(x_ref, wdkv_ref, wuk_ref, wuv_ref, wq_ref, wqr_ref, wkr_ref,
             wo_ref, out_ref, kp_ref, vp_ref, krecv_ref, vrecv_ref,
             send_sems, recv_sems):
        my_x = lax.axis_index("x")
        my_y = lax.axis_index("y")
        my_z = lax.axis_index("z")
        peer = (my_x, 1 - my_y, my_z)

        barrier = pltpu.get_barrier_semaphore()
        pl.semaphore_signal(barrier, inc=1, device_id=peer,
                            device_id_type=pl.DeviceIdType.MESH)
        pl.semaphore_wait(barrier, 1)

        xv = x_ref[...]
        c = jnp.dot(xv, wdkv_ref[...], preferred_element_type=F32)
        c = c.astype(BF16)
        kp_ref[...] = jnp.dot(c, wuk_ref[...],
                              preferred_element_type=F32).astype(BF16)
        vp_ref[...] = jnp.dot(c, wuv_ref[...],
                              preferred_element_type=F32).astype(BF16)

        rdma_k = pltpu.make_async_remote_copy(
            src_ref=kp_ref, dst_ref=krecv_ref,
            send_sem=send_sems.at[0], recv_sem=recv_sems.at[0],
            device_id=peer, device_id_type=pl.DeviceIdType.MESH)
        rdma_k.start()
        rdma_v = pltpu.make_async_remote_copy(
            src_ref=vp_ref, dst_ref=vrecv_ref,
            send_sem=send_sems.at[1], recv_sem=recv_sems.at[1],
            device_id=peer, device_id_type=pl.DeviceIdType.MESH)
        rdma_v.start()

        q = jnp.dot(xv, wq_ref[...], preferred_element_type=F32).astype(BF16)
        qr = jnp.dot(xv, wqr_ref[...], preferred_element_type=F32).astype(BF16)
        kr = jnp.dot(xv, wkr_ref[...], preferred_element_type=F32).astype(BF16)

        rdma_k.wait()
        rdma_v.wait()

        k = (kp_ref[...].astype(F32) + krecv_ref[...].astype(F32)).astype(BF16)
        v = (vp_ref[...].astype(F32) + vrecv_ref[...].astype(F32)).astype(BF16)

        outs = []
        for h in range(H):
            q_h = q[:, h * DH:(h + 1) * DH]
            k_h = k[:, h * DH:(h + 1) * DH]
            v_h = v[:, h * DH:(h + 1) * DH]
            qr_h = qr[:, h * DR:(h + 1) * DR]
            s1 = lax.dot_general(q_h, k_h, (((1,), (1,)), ((), ())),
                                 preferred_element_type=F32)
            s2 = lax.dot_general(qr_h, kr, (((1,), (1,)), ((), ())),
                                 preferred_element_type=F32)
            sc = (s1 + s2) * SCALE
            m = jnp.max(sc, axis=1, keepdims=True)
            p = jnp.exp(sc - m)
            p = (p / jnp.sum(p, axis=1, keepdims=True)).astype(BF16)
            outs.append(jnp.dot(p, v_h,
                                preferred_element_type=F32).astype(BF16))
        o = jnp.concatenate(outs, axis=1)
        out_ref[...] = jnp.dot(o, wo_ref[...], preferred_element_type=F32)

    out = pl.pallas_call(
        body,
        out_shape=jax.ShapeDtypeStruct((S, D), F32),
        in_specs=[pl.BlockSpec(memory_space=pltpu.VMEM)] * 8,
        out_specs=pl.BlockSpec(memory_space=pltpu.VMEM),
        scratch_shapes=[
            pltpu.VMEM((S, D), BF16),
            pltpu.VMEM((S, D), BF16),
            pltpu.VMEM((S, D), BF16),
            pltpu.VMEM((S, D), BF16),
            pltpu.SemaphoreType.DMA((2,)),
            pltpu.SemaphoreType.DMA((2,)),
        ],
        compiler_params=pltpu.CompilerParams(
            collective_id=0, vmem_limit_bytes=100 * 1024 * 1024),
    )(xb, wdkv, wuk, wuv, wq, wqr, wkr, wo)
    return out.reshape(1, S, D)


# device time: 119861 ns/iter; 1.6660x vs baseline; 1.6660x over previous
import jax
import jax.numpy as jnp
from jax import lax
from jax.experimental import pallas as pl
from jax.experimental.pallas import tpu as pltpu

S = 1024
D = 2048
H = 16
DH = 128
DR = 32
DC = 128
HB = (H // 2) * DH
SCALE = (DH + DR) ** -0.5
BF16 = jnp.bfloat16
F32 = jnp.float32


def kernel(x, Wdkv, Wuk, Wuv, Wq, Wqr, Wkr, Wo):
    xb = x.reshape(S, D).astype(BF16)
    wdkv = Wdkv.astype(BF16)
    wuk = Wuk.astype(BF16)
    wuv = Wuv.astype(BF16)
    wq = Wq.astype(BF16)
    wqr = Wqr.astype(BF16)
    wkr = Wkr.astype(BF16)
    wo = Wo.astype(BF16)

    def body(x_ref, wdkv_ref, wuk_ref, wuv_ref, wq_ref, wqr_ref, wkr_ref,
             wo_ref, out_ref, c_send, c_recv, wuk_recv, wuv_recv,
             o_send, o_recv, send_sems, recv_sems):
        my_x = lax.axis_index("x")
        my_y = lax.axis_index("y")
        my_z = lax.axis_index("z")
        peer = (my_x, 1 - my_y, my_z)
        my_col = my_y * HB
        peer_col = (1 - my_y) * HB

        barrier = pltpu.get_barrier_semaphore()
        pl.semaphore_signal(barrier, inc=1, device_id=peer,
                            device_id_type=pl.DeviceIdType.MESH)
        pl.semaphore_wait(barrier, 1)

        def rdma(src, dst, i):
            return pltpu.make_async_remote_copy(
                src_ref=src, dst_ref=dst,
                send_sem=send_sems.at[i], recv_sem=recv_sems.at[i],
                device_id=peer, device_id_type=pl.DeviceIdType.MESH)

        rdma_wuk = rdma(wuk_ref.at[:, pl.ds(peer_col, HB)], wuk_recv, 0)
        rdma_wuv = rdma(wuv_ref.at[:, pl.ds(peer_col, HB)], wuv_recv, 1)
        rdma_wuk.start()
        rdma_wuv.start()

        xv = x_ref[...]
        c = jnp.dot(xv, wdkv_ref[...], preferred_element_type=F32).astype(BF16)
        c_send[...] = c
        rdma_c = rdma(c_send, c_recv, 2)
        rdma_c.start()

        q = jnp.dot(xv, wq_ref[:, pl.ds(my_col, HB)],
                    preferred_element_type=F32).astype(BF16)
        qr = jnp.dot(xv, wqr_ref[:, pl.ds(my_y * (H // 2) * DR, (H // 2) * DR)],
                     preferred_element_type=F32).astype(BF16)
        kr = jnp.dot(xv, wkr_ref[...], preferred_element_type=F32).astype(BF16)

        rdma_wuk.wait()
        rdma_wuv.wait()
        rdma_c.wait()

        cp = c_recv[...]
        k = (jnp.dot(c, wuk_ref[:, pl.ds(my_col, HB)],
                     preferred_element_type=F32)
             + jnp.dot(cp, wuk_recv[...],
                       preferred_element_type=F32)).astype(BF16)
        v = (jnp.dot(c, wuv_ref[:, pl.ds(my_col, HB)],
                     preferred_element_type=F32)
             + jnp.dot(cp, wuv_recv[...],
                       preferred_element_type=F32)).astype(BF16)

        for h in range(H // 2):
            q_h = q[:, h * DH:(h + 1) * DH]
            k_h = k[:, h * DH:(h + 1) * DH]
            v_h = v[:, h * DH:(h + 1) * DH]
            qr_h = qr[:, h * DR:(h + 1) * DR]
            s1 = lax.dot_general(q_h, k_h, (((1,), (1,)), ((), ())),
                                 preferred_element_type=F32)
            s2 = lax.dot_general(qr_h, kr, (((1,), (1,)), ((), ())),
                                 preferred_element_type=F32)
            sc = (s1 + s2) * SCALE
            m = jnp.max(sc, axis=1, keepdims=True)
            p = jnp.exp(sc - m)
            p = (p / jnp.sum(p, axis=1, keepdims=True)).astype(BF16)
            o_send[:, h * DH:(h + 1) * DH] = jnp.dot(
                p, v_h, preferred_element_type=F32).astype(BF16)

        rdma_o = rdma(o_send, o_recv, 3)
        rdma_o.start()
        acc = jnp.dot(o_send[...], wo_ref[pl.ds(my_col, HB), :],
                      preferred_element_type=F32)
        rdma_o.wait()
        out_ref[...] = acc + jnp.dot(o_recv[...],
                                     wo_ref[pl.ds(peer_col, HB), :],
                                     preferred_element_type=F32)

    out = pl.pallas_call(
        body,
        out_shape=jax.ShapeDtypeStruct((S, D), F32),
        in_specs=[pl.BlockSpec(memory_space=pltpu.VMEM)] * 8,
        out_specs=pl.BlockSpec(memory_space=pltpu.VMEM),
        scratch_shapes=[
            pltpu.VMEM((S, DC), BF16),
            pltpu.VMEM((S, DC), BF16),
            pltpu.VMEM((DC, HB), BF16),
            pltpu.VMEM((DC, HB), BF16),
            pltpu.VMEM((S, HB), BF16),
            pltpu.VMEM((S, HB), BF16),
            pltpu.SemaphoreType.DMA((4,)),
            pltpu.SemaphoreType.DMA((4,)),
        ],
        compiler_params=pltpu.CompilerParams(
            collective_id=0, vmem_limit_bytes=100 * 1024 * 1024),
    )(xb, wdkv, wuk, wuv, wq, wqr, wkr, wo)
    return out.reshape(1, S, D)


# device time: 98818 ns/iter; 2.0207x vs baseline; 1.2129x over previous
import jax
import jax.numpy as jnp
from jax import lax
from jax.experimental import pallas as pl
from jax.experimental.pallas import tpu as pltpu

S = 1024
D = 2048
H = 16
DH = 128
DR = 32
DC = 128
HB = (H // 2) * DH
HC = HB // 2
SCALE = (DH + DR) ** -0.5
BF16 = jnp.bfloat16
F32 = jnp.float32


def kernel(x, Wdkv, Wuk, Wuv, Wq, Wqr, Wkr, Wo):
    my_y_out = lax.axis_index("y")
    xb = x.reshape(S, D).astype(BF16)
    wdkv = Wdkv.astype(BF16)
    wuk = Wuk.astype(BF16)
    wuv = Wuv.astype(BF16)
    wq = lax.dynamic_slice_in_dim(Wq, my_y_out * HB, HB, 1).astype(BF16)
    wqr = lax.dynamic_slice_in_dim(
        Wqr, my_y_out * (H // 2) * DR, (H // 2) * DR, 1).astype(BF16)
    wkr = Wkr.astype(BF16)
    wo = Wo.astype(BF16)

    def body(x_ref, wdkv_ref, wuk_ref, wuv_ref, wq_ref, wqr_ref, wkr_ref,
             wo_ref, out_ref, c_send, c_recv, wuk_recv, wuv_recv,
             o_send, o_recv, send_sems, recv_sems):
        my_x = lax.axis_index("x")
        my_y = lax.axis_index("y")
        my_z = lax.axis_index("z")
        peer = (my_x, 1 - my_y, my_z)
        my_col = my_y * HB
        peer_col = (1 - my_y) * HB

        barrier = pltpu.get_barrier_semaphore()
        pl.semaphore_signal(barrier, inc=1, device_id=peer,
                            device_id_type=pl.DeviceIdType.MESH)
        pl.semaphore_wait(barrier, 1)

        def rdma(src, dst, i):
            return pltpu.make_async_remote_copy(
                src_ref=src, dst_ref=dst,
                send_sem=send_sems.at[i], recv_sem=recv_sems.at[i],
                device_id=peer, device_id_type=pl.DeviceIdType.MESH)

        rdma_wuk = rdma(wuk_ref.at[:, pl.ds(peer_col, HB)], wuk_recv, 0)
        rdma_wuv = rdma(wuv_ref.at[:, pl.ds(peer_col, HB)], wuv_recv, 1)
        rdma_wuk.start()
        rdma_wuv.start()

        xv = x_ref[...]
        c = jnp.dot(xv, wdkv_ref[...], preferred_element_type=F32).astype(BF16)
        c_send[...] = c
        rdma_c = rdma(c_send, c_recv, 2)
        rdma_c.start()

        q = jnp.dot(xv, wq_ref[...], preferred_element_type=F32).astype(BF16)
        qr = jnp.dot(xv, wqr_ref[...], preferred_element_type=F32).astype(BF16)
        kr = jnp.dot(xv, wkr_ref[...], preferred_element_type=F32).astype(BF16)

        rdma_wuk.wait()
        rdma_wuv.wait()
        rdma_c.wait()

        cp = c_recv[...]
        k = (jnp.dot(c, wuk_ref[:, pl.ds(my_col, HB)],
                     preferred_element_type=F32)
             + jnp.dot(cp, wuk_recv[...],
                       preferred_element_type=F32)).astype(BF16)
        v = (jnp.dot(c, wuv_ref[:, pl.ds(my_col, HB)],
                     preferred_element_type=F32)
             + jnp.dot(cp, wuv_recv[...],
                       preferred_element_type=F32)).astype(BF16)

        o_rdmas = []
        for g in range(2):
            for hh in range(H // 4):
                h = g * (H // 4) + hh
                q_h = q[:, h * DH:(h + 1) * DH]
                k_h = k[:, h * DH:(h + 1) * DH]
                v_h = v[:, h * DH:(h + 1) * DH]
                qr_h = qr[:, h * DR:(h + 1) * DR]
                s1 = lax.dot_general(q_h, k_h, (((1,), (1,)), ((), ())),
                                     preferred_element_type=F32)
                s2 = lax.dot_general(qr_h, kr, (((1,), (1,)), ((), ())),
                                     preferred_element_type=F32)
                p = jnp.exp((s1 + s2) * SCALE)
                p = (p / jnp.sum(p, axis=1, keepdims=True)).astype(BF16)
                o_send[:, h * DH:(h + 1) * DH] = jnp.dot(
                    p, v_h, preferred_element_type=F32).astype(BF16)
            r = rdma(o_send.at[:, pl.ds(g * HC, HC)],
                     o_recv.at[:, pl.ds(g * HC, HC)], 3 + g)
            r.start()
            o_rdmas.append(r)

        acc = (jnp.dot(o_send[:, :HC], wo_ref[pl.ds(my_col, HC), :],
                       preferred_element_type=F32)
               + jnp.dot(o_send[:, HC:], wo_ref[pl.ds(my_col + HC, HC), :],
                         preferred_element_type=F32))
        o_rdmas[0].wait()
        acc = acc + jnp.dot(o_recv[:, :HC], wo_ref[pl.ds(peer_col, HC), :],
                            preferred_element_type=F32)
        o_rdmas[1].wait()
        out_ref[...] = acc + jnp.dot(o_recv[:, HC:],
                                     wo_ref[pl.ds(peer_col + HC, HC), :],
                                     preferred_element_type=F32)

    out = pl.pallas_call(
        body,
        out_shape=jax.ShapeDtypeStruct((S, D), F32),
        in_specs=[pl.BlockSpec(memory_space=pltpu.VMEM)] * 8,
        out_specs=pl.BlockSpec(memory_space=pltpu.VMEM),
        scratch_shapes=[
            pltpu.VMEM((S, DC), BF16),
            pltpu.VMEM((S, DC), BF16),
            pltpu.VMEM((DC, HB), BF16),
            pltpu.VMEM((DC, HB), BF16),
            pltpu.VMEM((S, HB), BF16),
            pltpu.VMEM((S, HB), BF16),
            pltpu.SemaphoreType.DMA((5,)),
            pltpu.SemaphoreType.DMA((5,)),
        ],
        compiler_params=pltpu.CompilerParams(
            collective_id=0, vmem_limit_bytes=63 * 1024 * 1024),
    )(xb, wdkv, wuk, wuv, wq, wqr, wkr, wo)
    return out.reshape(1, S, D)


# device time: 92743 ns/iter; 2.1531x vs baseline; 1.0655x over previous
import jax
import jax.numpy as jnp
from jax import lax
from jax.experimental import pallas as pl
from jax.experimental.pallas import tpu as pltpu

S = 1024
D = 2048
H = 16
DH = 128
DR = 32
DC = 128
HB = (H // 2) * DH
HC = HB // 2
RC = (H // 4) * DR
SCALE = (DH + DR) ** -0.5
BF16 = jnp.bfloat16
F32 = jnp.float32


def kernel(x, Wdkv, Wuk, Wuv, Wq, Wqr, Wkr, Wo):
    ix = lax.axis_index("x")
    iy = lax.axis_index("y")
    xb = x.reshape(S, D).astype(BF16)
    wdkv = Wdkv.astype(BF16)
    wuk = Wuk.astype(BF16)
    wuv = Wuv.astype(BF16)
    wq = lax.dynamic_slice_in_dim(Wq, iy * HB + ix * HC, HC, 1).astype(BF16)
    wqr = lax.dynamic_slice_in_dim(
        Wqr, iy * (H // 2) * DR + ix * RC, RC, 1).astype(BF16)
    wkr = Wkr.astype(BF16)
    wo = Wo.astype(BF16)

    def body(x_ref, wdkv_ref, wuk_ref, wuv_ref, wq_ref, wqr_ref, wkr_ref,
             wo_ref, out_ref, c_send, c_recv, wuk_recv, wuv_recv,
             o_mine, o_xnbr, o_peer, o_diag, send_sems, recv_sems):
        my_x = lax.axis_index("x")
        my_y = lax.axis_index("y")
        my_z = lax.axis_index("z")
        ypeer = (my_x, 1 - my_y, my_z)
        xnbr = (1 - my_x, my_y, my_z)

        qbase = my_y * HB + my_x * HC
        xn_base = my_y * HB + (1 - my_x) * HC
        pbase = (1 - my_y) * HB + my_x * HC
        dbase = (1 - my_y) * HB + (1 - my_x) * HC

        barrier = pltpu.get_barrier_semaphore()
        for nbr in (ypeer, xnbr):
            pl.semaphore_signal(barrier, inc=1, device_id=nbr,
                                device_id_type=pl.DeviceIdType.MESH)
        pl.semaphore_wait(barrier, 2)

        def rdma(src, dst, i, dev):
            return pltpu.make_async_remote_copy(
                src_ref=src, dst_ref=dst,
                send_sem=send_sems.at[i], recv_sem=recv_sems.at[i],
                device_id=dev, device_id_type=pl.DeviceIdType.MESH)

        rdma_wuk = rdma(wuk_ref.at[:, pl.ds(pbase, HC)], wuk_recv, 0, ypeer)
        rdma_wuv = rdma(wuv_ref.at[:, pl.ds(pbase, HC)], wuv_recv, 1, ypeer)
        rdma_wuk.start()
        rdma_wuv.start()

        xv = x_ref[...]
        c = jnp.dot(xv, wdkv_ref[...], preferred_element_type=F32).astype(BF16)
        c_send[...] = c
        rdma_c = rdma(c_send, c_recv, 2, ypeer)
        rdma_c.start()

        q = jnp.dot(xv, wq_ref[...], preferred_element_type=F32).astype(BF16)
        qr = jnp.dot(xv, wqr_ref[...], preferred_element_type=F32).astype(BF16)
        kr = jnp.dot(xv, wkr_ref[...], preferred_element_type=F32).astype(BF16)

        rdma_wuk.wait()
        rdma_wuv.wait()
        rdma_c.wait()

        cp = c_recv[...]
        k = (jnp.dot(c, wuk_ref[:, pl.ds(qbase, HC)],
                     preferred_element_type=F32)
             + jnp.dot(cp, wuk_recv[...],
                       preferred_element_type=F32)).astype(BF16)
        v = (jnp.dot(c, wuv_ref[:, pl.ds(qbase, HC)],
                     preferred_element_type=F32)
             + jnp.dot(cp, wuv_recv[...],
                       preferred_element_type=F32)).astype(BF16)

        for h in range(H // 4):
            q_h = q[:, h * DH:(h + 1) * DH]
            k_h = k[:, h * DH:(h + 1) * DH]
            v_h = v[:, h * DH:(h + 1) * DH]
            qr_h = qr[:, h * DR:(h + 1) * DR]
            s1 = lax.dot_general(q_h, k_h, (((1,), (1,)), ((), ())),
                                 preferred_element_type=F32)
            s2 = lax.dot_general(qr_h, kr, (((1,), (1,)), ((), ())),
                                 preferred_element_type=F32)
            p = jnp.exp((s1 + s2) * SCALE)
            p = (p / jnp.sum(p, axis=1, keepdims=True)).astype(BF16)
            o_mine[:, h * DH:(h + 1) * DH] = jnp.dot(
                p, v_h, preferred_element_type=F32).astype(BF16)

        rdma_ox = rdma(o_mine, o_xnbr, 5, xnbr)
        rdma_ox.start()
        rdma_op = rdma(o_mine, o_peer, 3, ypeer)
        rdma_op.start()

        acc = jnp.dot(o_mine[...], wo_ref[pl.ds(qbase, HC), :],
                      preferred_element_type=F32)

        rdma_ox.wait()
        rdma_of = rdma(o_xnbr, o_diag, 4, ypeer)
        rdma_of.start()
        acc = acc + jnp.dot(o_xnbr[...], wo_ref[pl.ds(xn_base, HC), :],
                            preferred_element_type=F32)

        rdma_op.wait()
        acc = acc + jnp.dot(o_peer[...], wo_ref[pl.ds(pbase, HC), :],
                            preferred_element_type=F32)
        rdma_of.wait()
        out_ref[...] = acc + jnp.dot(o_diag[...],
                                     wo_ref[pl.ds(dbase, HC), :],
                                     preferred_element_type=F32)

    out = pl.pallas_call(
        body,
        out_shape=jax.ShapeDtypeStruct((S, D), F32),
        in_specs=[pl.BlockSpec(memory_space=pltpu.VMEM)] * 8,
        out_specs=pl.BlockSpec(memory_space=pltpu.VMEM),
        scratch_shapes=[
            pltpu.VMEM((S, DC), BF16),
            pltpu.VMEM((S, DC), BF16),
            pltpu.VMEM((DC, HC), BF16),
            pltpu.VMEM((DC, HC), BF16),
            pltpu.VMEM((S, HC), BF16),
            pltpu.VMEM((S, HC), BF16),
            pltpu.VMEM((S, HC), BF16),
            pltpu.VMEM((S, HC), BF16),
            pltpu.SemaphoreType.DMA((6,)),
            pltpu.SemaphoreType.DMA((6,)),
        ],
        compiler_params=pltpu.CompilerParams(
            collective_id=0, vmem_limit_bytes=63 * 1024 * 1024),
    )(xb, wdkv, wuk, wuv, wq, wqr, wkr, wo)
    return out.reshape(1, S, D)


# device time: 79157 ns/iter; 2.5226x vs baseline; 1.1716x over previous
import jax
import jax.numpy as jnp
from jax import lax
from jax.experimental import pallas as pl
from jax.experimental.pallas import tpu as pltpu

S = 1024
D = 2048
H = 16
DH = 128
DR = 32
DC = 128
HB = (H // 2) * DH
HC = HB // 2
RC = (H // 4) * DR
SCALE = (DH + DR) ** -0.5
BF16 = jnp.bfloat16
F32 = jnp.float32


def kernel(x, Wdkv, Wuk, Wuv, Wq, Wqr, Wkr, Wo):
    ix = lax.axis_index("x")
    iy = lax.axis_index("y")
    xb = x.reshape(S, D).astype(BF16)
    wdkv = Wdkv.astype(BF16)
    wuk = Wuk.astype(BF16)
    wuv = Wuv.astype(BF16)
    wqr = lax.dynamic_slice_in_dim(
        Wqr, iy * (H // 2) * DR + ix * RC, RC, 1).astype(BF16)
    wkr = Wkr.astype(BF16)

    def body(x_ref, wdkv_ref, wuk_ref, wuv_ref, wq_hbm, wqr_ref, wkr_ref,
             wo_hbm, out_ref, c_send, c_recv, wuk_recv, wuv_recv,
             o_mine, o_xnbr, o_peer, o_diag, wq_buf, wo_buf,
             send_sems, recv_sems, local_sems):
        my_x = lax.axis_index("x")
        my_y = lax.axis_index("y")
        my_z = lax.axis_index("z")
        ypeer = (my_x, 1 - my_y, my_z)
        xnbr = (1 - my_x, my_y, my_z)

        qbase = my_y * HB + my_x * HC
        xn_base = my_y * HB + (1 - my_x) * HC
        pbase = (1 - my_y) * HB + my_x * HC
        dbase = (1 - my_y) * HB + (1 - my_x) * HC

        cp_wq = pltpu.make_async_copy(
            wq_hbm.at[:, pl.ds(qbase, HC)], wq_buf, local_sems.at[0])
        cp_wq.start()
        cp_wo0 = pltpu.make_async_copy(
            wo_hbm.at[pl.ds(qbase, HC), :], wo_buf.at[0], local_sems.at[1])
        cp_wo0.start()
        cp_wo1 = pltpu.make_async_copy(
            wo_hbm.at[pl.ds(xn_base, HC), :], wo_buf.at[1], local_sems.at[2])
        cp_wo1.start()

        barrier = pltpu.get_barrier_semaphore()
        for nbr in (ypeer, xnbr):
            pl.semaphore_signal(barrier, inc=1, device_id=nbr,
                                device_id_type=pl.DeviceIdType.MESH)
        pl.semaphore_wait(barrier, 2)

        def rdma(src, dst, i, dev):
            return pltpu.make_async_remote_copy(
                src_ref=src, dst_ref=dst,
                send_sem=send_sems.at[i], recv_sem=recv_sems.at[i],
                device_id=dev, device_id_type=pl.DeviceIdType.MESH)

        rdma_wuk = rdma(wuk_ref.at[:, pl.ds(pbase, HC)], wuk_recv, 0, ypeer)
        rdma_wuv = rdma(wuv_ref.at[:, pl.ds(pbase, HC)], wuv_recv, 1, ypeer)
        rdma_wuk.start()
        rdma_wuv.start()

        xv = x_ref[...]
        c = jnp.dot(xv, wdkv_ref[...], preferred_element_type=F32).astype(BF16)
        c_send[...] = c
        rdma_c = rdma(c_send, c_recv, 2, ypeer)
        rdma_c.start()

        cp_wq.wait()
        q = jnp.dot(xv, wq_buf[...].astype(BF16),
                    preferred_element_type=F32).astype(BF16)
        qr = jnp.dot(xv, wqr_ref[...], preferred_element_type=F32).astype(BF16)
        kr = jnp.dot(xv, wkr_ref[...], preferred_element_type=F32).astype(BF16)

        rdma_wuk.wait()
        rdma_wuv.wait()
        rdma_c.wait()

        cp = c_recv[...]
        k = (jnp.dot(c, wuk_ref[:, pl.ds(qbase, HC)],
                     preferred_element_type=F32)
             + jnp.dot(cp, wuk_recv[...],
                       preferred_element_type=F32)).astype(BF16)
        v = (jnp.dot(c, wuv_ref[:, pl.ds(qbase, HC)],
                     preferred_element_type=F32)
             + jnp.dot(cp, wuv_recv[...],
                       preferred_element_type=F32)).astype(BF16)

        for h in range(H // 4):
            q_h = q[:, h * DH:(h + 1) * DH]
            k_h = k[:, h * DH:(h + 1) * DH]
            v_h = v[:, h * DH:(h + 1) * DH]
            qr_h = qr[:, h * DR:(h + 1) * DR]
            s1 = lax.dot_general(q_h, k_h, (((1,), (1,)), ((), ())),
                                 preferred_element_type=F32)
            s2 = lax.dot_general(qr_h, kr, (((1,), (1,)), ((), ())),
                                 preferred_element_type=F32)
            p = jnp.exp((s1 + s2) * SCALE)
            p = (p / jnp.sum(p, axis=1, keepdims=True)).astype(BF16)
            o_mine[:, h * DH:(h + 1) * DH] = jnp.dot(
                p, v_h, preferred_element_type=F32).astype(BF16)

        rdma_ox = rdma(o_mine, o_xnbr, 5, xnbr)
        rdma_ox.start()
        rdma_op = rdma(o_mine, o_peer, 3, ypeer)
        rdma_op.start()

        cp_wo0.wait()
        acc = jnp.dot(o_mine[...], wo_buf[0].astype(BF16),
                      preferred_element_type=F32)
        cp_wo2 = pltpu.make_async_copy(
            wo_hbm.at[pl.ds(pbase, HC), :], wo_buf.at[0], local_sems.at[1])
        cp_wo2.start()

        rdma_ox.wait()
        rdma_of = rdma(o_xnbr, o_diag, 4, ypeer)
        rdma_of.start()
        cp_wo1.wait()
        acc = acc + jnp.dot(o_xnbr[...], wo_buf[1].astype(BF16),
                            preferred_element_type=F32)
        cp_wo3 = pltpu.make_async_copy(
            wo_hbm.at[pl.ds(dbase, HC), :], wo_buf.at[1], local_sems.at[2])
        cp_wo3.start()

        rdma_op.wait()
        cp_wo2.wait()
        acc = acc + jnp.dot(o_peer[...], wo_buf[0].astype(BF16),
                            preferred_element_type=F32)
        rdma_of.wait()
        cp_wo3.wait()
        out_ref[...] = acc + jnp.dot(o_diag[...], wo_buf[1].astype(BF16),
                                     preferred_element_type=F32)

    in_specs = [pl.BlockSpec(memory_space=pltpu.VMEM)] * 8
    in_specs[4] = pl.BlockSpec(memory_space=pl.ANY)
    in_specs[7] = pl.BlockSpec(memory_space=pl.ANY)

    out = pl.pallas_call(
        body,
        out_shape=jax.ShapeDtypeStruct((S, D), F32),
        in_specs=in_specs,
        out_specs=pl.BlockSpec(memory_space=pltpu.VMEM),
        scratch_shapes=[
            pltpu.VMEM((S, DC), BF16),
            pltpu.VMEM((S, DC), BF16),
            pltpu.VMEM((DC, HC), BF16),
            pltpu.VMEM((DC, HC), BF16),
            pltpu.VMEM((S, HC), BF16),
            pltpu.VMEM((S, HC), BF16),
            pltpu.VMEM((S, HC), BF16),
            pltpu.VMEM((S, HC), BF16),
            pltpu.VMEM((D, HC), F32),
            pltpu.VMEM((2, HC, D), F32),
            pltpu.SemaphoreType.DMA((6,)),
            pltpu.SemaphoreType.DMA((6,)),
            pltpu.SemaphoreType.DMA((3,)),
        ],
        compiler_params=pltpu.CompilerParams(
            collective_id=0, vmem_limit_bytes=63 * 1024 * 1024),
    )(xb, wdkv, wuk, wuv, Wq, wqr, wkr, Wo)
    return out.reshape(1, S, D)


# device time: 76194 ns/iter; 2.6207x vs baseline; 1.0389x over previous
import jax
import jax.numpy as jnp
from jax import lax
from jax.experimental import pallas as pl
from jax.experimental.pallas import tpu as pltpu

S = 1024
D = 2048
H = 16
DH = 128
DR = 32
DC = 128
HB = (H // 2) * DH
HC = HB // 2
RC = (H // 4) * DR
SCALE = (DH + DR) ** -0.5
BF16 = jnp.bfloat16
F32 = jnp.float32


def kernel(x, Wdkv, Wuk, Wuv, Wq, Wqr, Wkr, Wo):
    ix = lax.axis_index("x")
    iy = lax.axis_index("y")
    xb = x.reshape(S, D).astype(BF16)
    wdkv = Wdkv.astype(BF16)
    wuk = Wuk.astype(BF16)
    wuv = Wuv.astype(BF16)
    wqr = lax.dynamic_slice_in_dim(
        Wqr, iy * (H // 2) * DR + ix * RC, RC, 1).astype(BF16)
    wkr = Wkr.astype(BF16)

    def body(x_ref, wdkv_ref, wuk_ref, wuv_ref, wq_hbm, wqr_ref, wkr_ref,
             wo_hbm, out_ref, c_send, c_recv, wuk_recv, wuv_recv,
             o_mine, o_xnbr, o_peer, o_diag, wq_buf, wo_buf,
             send_sems, recv_sems, local_sems):
        my_x = lax.axis_index("x")
        my_y = lax.axis_index("y")
        my_z = lax.axis_index("z")
        ypeer = (my_x, 1 - my_y, my_z)
        xnbr = (1 - my_x, my_y, my_z)

        qbase = my_y * HB + my_x * HC
        xn_base = my_y * HB + (1 - my_x) * HC
        pbase = (1 - my_y) * HB + my_x * HC
        dbase = (1 - my_y) * HB + (1 - my_x) * HC

        cp_wq = pltpu.make_async_copy(
            wq_hbm.at[:, pl.ds(qbase, HC)], wq_buf, local_sems.at[0])
        cp_wq.start()
        cp_wo0 = pltpu.make_async_copy(
            wo_hbm.at[pl.ds(qbase, HC), :], wo_buf.at[0], local_sems.at[1])
        cp_wo0.start()
        cp_wo1 = pltpu.make_async_copy(
            wo_hbm.at[pl.ds(xn_base, HC), :], wo_buf.at[1], local_sems.at[2])
        cp_wo1.start()

        barrier = pltpu.get_barrier_semaphore()
        for nbr in (ypeer, xnbr):
            pl.semaphore_signal(barrier, inc=1, device_id=nbr,
                                device_id_type=pl.DeviceIdType.MESH)
        pl.semaphore_wait(barrier, 2)

        def rdma(src, dst, i, dev):
            return pltpu.make_async_remote_copy(
                src_ref=src, dst_ref=dst,
                send_sem=send_sems.at[i], recv_sem=recv_sems.at[i],
                device_id=dev, device_id_type=pl.DeviceIdType.MESH)

        rdma_wuk = rdma(wuk_ref.at[:, pl.ds(pbase, HC)], wuk_recv, 0, ypeer)
        rdma_wuv = rdma(wuv_ref.at[:, pl.ds(pbase, HC)], wuv_recv, 1, ypeer)
        rdma_wuk.start()
        rdma_wuv.start()

        xv = x_ref[...]
        c = jnp.dot(xv, wdkv_ref[...], preferred_element_type=F32).astype(BF16)
        c_send[...] = c
        rdma_c = rdma(c_send, c_recv, 2, ypeer)
        rdma_c.start()

        cp_wq.wait()
        q = jnp.dot(xv, wq_buf[...].astype(BF16),
                    preferred_element_type=F32).astype(BF16)
        qr = jnp.dot(xv, wqr_ref[...], preferred_element_type=F32).astype(BF16)
        kr = jnp.dot(xv, wkr_ref[...], preferred_element_type=F32).astype(BF16)

        rdma_wuk.wait()
        rdma_wuv.wait()
        rdma_c.wait()

        cp = c_recv[...]
        k = (jnp.dot(c, wuk_ref[:, pl.ds(qbase, HC)],
                     preferred_element_type=F32)
             + jnp.dot(cp, wuk_recv[...],
                       preferred_element_type=F32)).astype(BF16)
        v = (jnp.dot(c, wuv_ref[:, pl.ds(qbase, HC)],
                     preferred_element_type=F32)
             + jnp.dot(cp, wuv_recv[...],
                       preferred_element_type=F32)).astype(BF16)

        qs = (q.astype(F32) * SCALE).astype(BF16)
        qrs = (qr.astype(F32) * SCALE).astype(BF16)
        for h in range(H // 4):
            qcat = jnp.concatenate(
                [qs[:, h * DH:(h + 1) * DH], qrs[:, h * DR:(h + 1) * DR]],
                axis=1)
            kcat = jnp.concatenate([k[:, h * DH:(h + 1) * DH], kr], axis=1)
            v_h = v[:, h * DH:(h + 1) * DH]
            p = jnp.exp(lax.dot_general(qcat, kcat, (((1,), (1,)), ((), ())),
                                        preferred_element_type=F32))
            rs = jnp.sum(p, axis=1, keepdims=True)
            o_h = jnp.dot(p.astype(BF16), v_h, preferred_element_type=F32)
            o_mine[:, h * DH:(h + 1) * DH] = (o_h / rs).astype(BF16)

        rdma_ox = rdma(o_mine, o_xnbr, 5, xnbr)
        rdma_ox.start()
        rdma_op = rdma(o_mine, o_peer, 3, ypeer)
        rdma_op.start()

        cp_wo0.wait()
        acc = jnp.dot(o_mine[...], wo_buf[0].astype(BF16),
                      preferred_element_type=F32)
        cp_wo2 = pltpu.make_async_copy(
            wo_hbm.at[pl.ds(pbase, HC), :], wo_buf.at[0], local_sems.at[1])
        cp_wo2.start()

        rdma_ox.wait()
        rdma_of = rdma(o_xnbr, o_diag, 4, ypeer)
        rdma_of.start()
        cp_wo1.wait()
        acc = acc + jnp.dot(o_xnbr[...], wo_buf[1].astype(BF16),
                            preferred_element_type=F32)
        cp_wo3 = pltpu.make_async_copy(
            wo_hbm.at[pl.ds(dbase, HC), :], wo_buf.at[1], local_sems.at[2])
        cp_wo3.start()

        rdma_op.wait()
        cp_wo2.wait()
        acc = acc + jnp.dot(o_peer[...], wo_buf[0].astype(BF16),
                            preferred_element_type=F32)
        rdma_of.wait()
        cp_wo3.wait()
        out_ref[...] = acc + jnp.dot(o_diag[...], wo_buf[1].astype(BF16),
                                     preferred_element_type=F32)

    in_specs = [pl.BlockSpec(memory_space=pltpu.VMEM)] * 8
    in_specs[4] = pl.BlockSpec(memory_space=pl.ANY)
    in_specs[7] = pl.BlockSpec(memory_space=pl.ANY)

    out = pl.pallas_call(
        body,
        out_shape=jax.ShapeDtypeStruct((S, D), F32),
        in_specs=in_specs,
        out_specs=pl.BlockSpec(memory_space=pltpu.VMEM),
        scratch_shapes=[
            pltpu.VMEM((S, DC), BF16),
            pltpu.VMEM((S, DC), BF16),
            pltpu.VMEM((DC, HC), BF16),
            pltpu.VMEM((DC, HC), BF16),
            pltpu.VMEM((S, HC), BF16),
            pltpu.VMEM((S, HC), BF16),
            pltpu.VMEM((S, HC), BF16),
            pltpu.VMEM((S, HC), BF16),
            pltpu.VMEM((D, HC), F32),
            pltpu.VMEM((2, HC, D), F32),
            pltpu.SemaphoreType.DMA((6,)),
            pltpu.SemaphoreType.DMA((6,)),
            pltpu.SemaphoreType.DMA((3,)),
        ],
        compiler_params=pltpu.CompilerParams(
            collective_id=0, vmem_limit_bytes=63 * 1024 * 1024),
    )(xb, wdkv, wuk, wuv, Wq, wqr, wkr, Wo)
    return out.reshape(1, S, D)


# device time: 65738 ns/iter; 3.0376x vs baseline; 1.1591x over previous
import jax
import jax.numpy as jnp
from jax import lax
from jax.experimental import pallas as pl
from jax.experimental.pallas import tpu as pltpu

S = 1024
D = 2048
H = 16
DH = 128
DR = 32
DC = 128
HB = (H // 2) * DH
HC = HB // 2
RC = (H // 4) * DR
SCALE = (DH + DR) ** -0.5
BF16 = jnp.bfloat16
F32 = jnp.float32


def kernel(x, Wdkv, Wuk, Wuv, Wq, Wqr, Wkr, Wo):
    def body(x_ref, wdkv_ref, wuk_ref, wuv_ref, wq_hbm, wqr_hbm, wkr_ref,
             wo_hbm, out_ref, c_send, c_recv, wuk_recv, wuv_recv,
             o_mine, o_xnbr, o_peer, o_diag, wq_buf, wqr_buf, wo_buf,
             send_sems, recv_sems, local_sems):
        my_x = lax.axis_index("x")
        my_y = lax.axis_index("y")
        my_z = lax.axis_index("z")
        ypeer = (my_x, 1 - my_y, my_z)
        xnbr = (1 - my_x, my_y, my_z)

        qbase = my_y * HB + my_x * HC
        xn_base = my_y * HB + (1 - my_x) * HC
        pbase = (1 - my_y) * HB + my_x * HC
        dbase = (1 - my_y) * HB + (1 - my_x) * HC

        cp_wq = pltpu.make_async_copy(
            wq_hbm.at[:, pl.ds(qbase, HC)], wq_buf, local_sems.at[0])
        cp_wq.start()
        cp_wqr = pltpu.make_async_copy(
            wqr_hbm.at[:, pl.ds(my_y * (H // 2) * DR + my_x * RC, RC)],
            wqr_buf, local_sems.at[3])
        cp_wqr.start()
        cp_wo0 = pltpu.make_async_copy(
            wo_hbm.at[pl.ds(qbase, HC), :], wo_buf.at[0], local_sems.at[1])
        cp_wo0.start()
        cp_wo1 = pltpu.make_async_copy(
            wo_hbm.at[pl.ds(xn_base, HC), :], wo_buf.at[1], local_sems.at[2])
        cp_wo1.start()

        barrier = pltpu.get_barrier_semaphore()
        for nbr in (ypeer, xnbr):
            pl.semaphore_signal(barrier, inc=1, device_id=nbr,
                                device_id_type=pl.DeviceIdType.MESH)
        pl.semaphore_wait(barrier, 2)

        def rdma(src, dst, i, dev):
            return pltpu.make_async_remote_copy(
                src_ref=src, dst_ref=dst,
                send_sem=send_sems.at[i], recv_sem=recv_sems.at[i],
                device_id=dev, device_id_type=pl.DeviceIdType.MESH)

        rdma_wuk = rdma(wuk_ref.at[:, pl.ds(pbase, HC)], wuk_recv, 0, ypeer)
        rdma_wuv = rdma(wuv_ref.at[:, pl.ds(pbase, HC)], wuv_recv, 1, ypeer)
        rdma_wuk.start()
        rdma_wuv.start()

        xv = x_ref[...].astype(BF16)
        c = jnp.dot(xv, wdkv_ref[...].astype(BF16),
                    preferred_element_type=F32).astype(BF16)
        c_send[...] = c
        rdma_c = rdma(c_send, c_recv, 2, ypeer)
        rdma_c.start()

        cp_wq.wait()
        q = jnp.dot(xv, wq_buf[...].astype(BF16),
                    preferred_element_type=F32).astype(BF16)
        cp_wqr.wait()
        qr = jnp.dot(xv, wqr_buf[...].astype(BF16),
                     preferred_element_type=F32).astype(BF16)
        kr = jnp.dot(xv, wkr_ref[...].astype(BF16),
                     preferred_element_type=F32).astype(BF16)

        rdma_wuk.wait()
        rdma_wuv.wait()
        rdma_c.wait()

        cp = c_recv[...]
        k = (jnp.dot(c, wuk_ref[:, pl.ds(qbase, HC)].astype(BF16),
                     preferred_element_type=F32)
             + jnp.dot(cp, wuk_recv[...].astype(BF16),
                       preferred_element_type=F32)).astype(BF16)
        v = (jnp.dot(c, wuv_ref[:, pl.ds(qbase, HC)].astype(BF16),
                     preferred_element_type=F32)
             + jnp.dot(cp, wuv_recv[...].astype(BF16),
                       preferred_element_type=F32)).astype(BF16)

        qs = (q.astype(F32) * SCALE).astype(BF16)
        qrs = (qr.astype(F32) * SCALE).astype(BF16)
        for h in range(H // 4):
            qcat = jnp.concatenate(
                [qs[:, h * DH:(h + 1) * DH], qrs[:, h * DR:(h + 1) * DR]],
                axis=1)
            kcat = jnp.concatenate([k[:, h * DH:(h + 1) * DH], kr], axis=1)
            v_h = v[:, h * DH:(h + 1) * DH]
            p = jnp.exp(lax.dot_general(qcat, kcat, (((1,), (1,)), ((), ())),
                                        preferred_element_type=F32))
            rs = jnp.sum(p, axis=1, keepdims=True)
            o_h = jnp.dot(p.astype(BF16), v_h, preferred_element_type=F32)
            o_mine[:, h * DH:(h + 1) * DH] = (o_h / rs).astype(BF16)

        rdma_ox = rdma(o_mine, o_xnbr, 5, xnbr)
        rdma_ox.start()
        rdma_op = rdma(o_mine, o_peer, 3, ypeer)
        rdma_op.start()

        cp_wo0.wait()
        acc = jnp.dot(o_mine[...], wo_buf[0].astype(BF16),
                      preferred_element_type=F32)
        cp_wo2 = pltpu.make_async_copy(
            wo_hbm.at[pl.ds(pbase, HC), :], wo_buf.at[0], local_sems.at[1])
        cp_wo2.start()

        rdma_ox.wait()
        rdma_of = rdma(o_xnbr, o_diag, 4, ypeer)
        rdma_of.start()
        cp_wo1.wait()
        acc = acc + jnp.dot(o_xnbr[...], wo_buf[1].astype(BF16),
                            preferred_element_type=F32)
        cp_wo3 = pltpu.make_async_copy(
            wo_hbm.at[pl.ds(dbase, HC), :], wo_buf.at[1], local_sems.at[2])
        cp_wo3.start()

        rdma_op.wait()
        cp_wo2.wait()
        acc = acc + jnp.dot(o_peer[...], wo_buf[0].astype(BF16),
                            preferred_element_type=F32)
        rdma_of.wait()
        cp_wo3.wait()
        out_ref[...] = acc + jnp.dot(o_diag[...], wo_buf[1].astype(BF16),
                                     preferred_element_type=F32)

    in_specs = [pl.BlockSpec(memory_space=pltpu.VMEM)] * 8
    in_specs[4] = pl.BlockSpec(memory_space=pl.ANY)
    in_specs[5] = pl.BlockSpec(memory_space=pl.ANY)
    in_specs[7] = pl.BlockSpec(memory_space=pl.ANY)

    out = pl.pallas_call(
        body,
        out_shape=jax.ShapeDtypeStruct((S, D), F32),
        in_specs=in_specs,
        out_specs=pl.BlockSpec(memory_space=pltpu.VMEM),
        scratch_shapes=[
            pltpu.VMEM((S, DC), BF16),
            pltpu.VMEM((S, DC), BF16),
            pltpu.VMEM((DC, HC), F32),
            pltpu.VMEM((DC, HC), F32),
            pltpu.VMEM((S, HC), BF16),
            pltpu.VMEM((S, HC), BF16),
            pltpu.VMEM((S, HC), BF16),
            pltpu.VMEM((S, HC), BF16),
            pltpu.VMEM((D, HC), F32),
            pltpu.VMEM((D, RC), F32),
            pltpu.VMEM((2, HC, D), F32),
            pltpu.SemaphoreType.DMA((6,)),
            pltpu.SemaphoreType.DMA((6,)),
            pltpu.SemaphoreType.DMA((4,)),
        ],
        compiler_params=pltpu.CompilerParams(
            collective_id=0, vmem_limit_bytes=63 * 1024 * 1024),
    )(x.reshape(S, D), Wdkv, Wuk, Wuv, Wq, Wqr, Wkr, Wo)
    return out.reshape(1, S, D)


# device time: 62134 ns/iter; 3.2138x vs baseline; 1.0580x over previous
import jax
import jax.numpy as jnp
from jax import lax
from jax.experimental import pallas as pl
from jax.experimental.pallas import tpu as pltpu

S = 1024
D = 2048
H = 16
DH = 128
DR = 32
DC = 128
HB = (H // 2) * DH
HC = HB // 2
RC = (H // 4) * DR
SCALE = (DH + DR) ** -0.5
BF16 = jnp.bfloat16
F32 = jnp.float32


def kernel(x, Wdkv, Wuk, Wuv, Wq, Wqr, Wkr, Wo):
    def body(x_ref, wdkv_ref, wuk_ref, wuv_ref, wq_hbm, wqr_hbm, wkr_ref,
             wo_hbm, out_ref, c_send, c_recv, wuk_recv, wuv_recv,
             o_mine, o_xnbr, o_peer, o_diag, wq_buf, wqr_buf, wo_buf,
             send_sems, recv_sems, local_sems):
        my_x = lax.axis_index("x")
        my_y = lax.axis_index("y")
        my_z = lax.axis_index("z")
        ypeer = (my_x, 1 - my_y, my_z)
        xnbr = (1 - my_x, my_y, my_z)

        qbase = my_y * HB + my_x * HC
        xn_base = my_y * HB + (1 - my_x) * HC
        pbase = (1 - my_y) * HB + my_x * HC
        dbase = (1 - my_y) * HB + (1 - my_x) * HC

        cp_wq = pltpu.make_async_copy(
            wq_hbm.at[:, pl.ds(qbase, HC)], wq_buf, local_sems.at[0])
        cp_wq.start()
        cp_wqr = pltpu.make_async_copy(
            wqr_hbm.at[:, pl.ds(my_y * (H // 2) * DR + my_x * RC, RC)],
            wqr_buf, local_sems.at[3])
        cp_wqr.start()
        cp_wo0 = pltpu.make_async_copy(
            wo_hbm.at[pl.ds(qbase, HC), :], wo_buf.at[0], local_sems.at[1])
        cp_wo0.start()
        cp_wo1 = pltpu.make_async_copy(
            wo_hbm.at[pl.ds(xn_base, HC), :], wo_buf.at[1], local_sems.at[2])
        cp_wo1.start()

        barrier = pltpu.get_barrier_semaphore()
        for nbr in (ypeer, xnbr):
            pl.semaphore_signal(barrier, inc=1, device_id=nbr,
                                device_id_type=pl.DeviceIdType.MESH)
        pl.semaphore_wait(barrier, 2)

        def rdma(src, dst, i, dev):
            return pltpu.make_async_remote_copy(
                src_ref=src, dst_ref=dst,
                send_sem=send_sems.at[i], recv_sem=recv_sems.at[i],
                device_id=dev, device_id_type=pl.DeviceIdType.MESH)

        rdma_wuk = rdma(wuk_ref.at[:, pl.ds(pbase, HC)], wuk_recv, 0, ypeer)
        rdma_wuv = rdma(wuv_ref.at[:, pl.ds(pbase, HC)], wuv_recv, 1, ypeer)
        rdma_wuk.start()
        rdma_wuv.start()

        xv = x_ref[...].astype(BF16)
        c = jnp.dot(xv, wdkv_ref[...].astype(BF16),
                    preferred_element_type=F32).astype(BF16)
        c_send[...] = c
        rdma_c = rdma(c_send, c_recv, 2, ypeer)
        rdma_c.start()

        cp_wq.wait()
        q = jnp.dot(xv, wq_buf[...].astype(BF16),
                    preferred_element_type=F32).astype(BF16)
        cp_wqr.wait()
        qr = jnp.dot(xv, wqr_buf[...].astype(BF16),
                     preferred_element_type=F32).astype(BF16)
        kr = jnp.dot(xv, wkr_ref[...].astype(BF16),
                     preferred_element_type=F32).astype(BF16)
        qs = (q.astype(F32) * SCALE).astype(BF16)
        qrs = (qr.astype(F32) * SCALE).astype(BF16)

        rdma_wuk.wait()
        rdma_wuv.wait()
        rdma_c.wait()

        cp = c_recv[...]
        k = (jnp.dot(c, wuk_ref[:, pl.ds(qbase, HC)].astype(BF16),
                     preferred_element_type=F32)
             + jnp.dot(cp, wuk_recv[...].astype(BF16),
                       preferred_element_type=F32)).astype(BF16)
        v = (jnp.dot(c, wuv_ref[:, pl.ds(qbase, HC)].astype(BF16),
                     preferred_element_type=F32)
             + jnp.dot(cp, wuv_recv[...].astype(BF16),
                       preferred_element_type=F32)).astype(BF16)

        OC = 2 * DH
        ox_rdmas, op_rdmas = [], []
        for h in range(H // 4):
            qcat = jnp.concatenate(
                [qs[:, h * DH:(h + 1) * DH], qrs[:, h * DR:(h + 1) * DR]],
                axis=1)
            kcat = jnp.concatenate([k[:, h * DH:(h + 1) * DH], kr], axis=1)
            v_h = v[:, h * DH:(h + 1) * DH]
            p = jnp.exp(lax.dot_general(qcat, kcat, (((1,), (1,)), ((), ())),
                                        preferred_element_type=F32))
            rs = jnp.sum(p, axis=1, keepdims=True)
            o_h = jnp.dot(p.astype(BF16), v_h, preferred_element_type=F32)
            o_mine[:, h * DH:(h + 1) * DH] = (o_h / rs).astype(BF16)
            if h % 2 == 1:
                g = h // 2
                sl = pl.ds(g * OC, OC)
                rx = rdma(o_mine.at[:, sl], o_xnbr.at[:, sl], 5 + g, xnbr)
                rx.start()
                ox_rdmas.append(rx)
                ry = rdma(o_mine.at[:, sl], o_peer.at[:, sl], 3 + g, ypeer)
                ry.start()
                op_rdmas.append(ry)

        cp_wo0.wait()
        acc = jnp.dot(o_mine[...], wo_buf[0].astype(BF16),
                      preferred_element_type=F32)
        cp_wo2 = pltpu.make_async_copy(
            wo_hbm.at[pl.ds(pbase, HC), :], wo_buf.at[0], local_sems.at[1])
        cp_wo2.start()

        of_rdmas = []
        for g in range(2):
            ox_rdmas[g].wait()
            sl = pl.ds(g * OC, OC)
            rf = rdma(o_xnbr.at[:, sl], o_diag.at[:, sl], 7 + g, ypeer)
            rf.start()
            of_rdmas.append(rf)
        cp_wo1.wait()
        acc = acc + jnp.dot(o_xnbr[...], wo_buf[1].astype(BF16),
                            preferred_element_type=F32)
        cp_wo3 = pltpu.make_async_copy(
            wo_hbm.at[pl.ds(dbase, HC), :], wo_buf.at[1], local_sems.at[2])
        cp_wo3.start()

        op_rdmas[0].wait()
        op_rdmas[1].wait()
        cp_wo2.wait()
        acc = acc + jnp.dot(o_peer[...], wo_buf[0].astype(BF16),
                            preferred_element_type=F32)
        of_rdmas[0].wait()
        of_rdmas[1].wait()
        cp_wo3.wait()
        out_ref[...] = acc + jnp.dot(o_diag[...], wo_buf[1].astype(BF16),
                                     preferred_element_type=F32)

    in_specs = [pl.BlockSpec(memory_space=pltpu.VMEM)] * 8
    in_specs[4] = pl.BlockSpec(memory_space=pl.ANY)
    in_specs[5] = pl.BlockSpec(memory_space=pl.ANY)
    in_specs[7] = pl.BlockSpec(memory_space=pl.ANY)

    out = pl.pallas_call(
        body,
        out_shape=jax.ShapeDtypeStruct((S, D), F32),
        in_specs=in_specs,
        out_specs=pl.BlockSpec(memory_space=pltpu.VMEM),
        scratch_shapes=[
            pltpu.VMEM((S, DC), BF16),
            pltpu.VMEM((S, DC), BF16),
            pltpu.VMEM((DC, HC), F32),
            pltpu.VMEM((DC, HC), F32),
            pltpu.VMEM((S, HC), BF16),
            pltpu.VMEM((S, HC), BF16),
            pltpu.VMEM((S, HC), BF16),
            pltpu.VMEM((S, HC), BF16),
            pltpu.VMEM((D, HC), F32),
            pltpu.VMEM((D, RC), F32),
            pltpu.VMEM((2, HC, D), F32),
            pltpu.SemaphoreType.DMA((9,)),
            pltpu.SemaphoreType.DMA((9,)),
            pltpu.SemaphoreType.DMA((4,)),
        ],
        compiler_params=pltpu.CompilerParams(
            collective_id=0, vmem_limit_bytes=63 * 1024 * 1024),
    )(x.reshape(S, D), Wdkv, Wuk, Wuv, Wq, Wqr, Wkr, Wo)
    return out.reshape(1, S, D)


# device time: 61193 ns/iter; 3.2632x vs baseline; 1.0154x over previous
import jax
import jax.numpy as jnp
from jax import lax
from jax.experimental import pallas as pl
from jax.experimental.pallas import tpu as pltpu

S = 1024
D = 2048
H = 16
DH = 128
DR = 32
DC = 128
HB = (H // 2) * DH
HC = HB // 2
RC = (H // 4) * DR
SCALE = (DH + DR) ** -0.5
BF16 = jnp.bfloat16
F32 = jnp.float32


def kernel(x, Wdkv, Wuk, Wuv, Wq, Wqr, Wkr, Wo):
    def body(x_ref, wdkv_ref, wuk_ref, wuv_ref, wq_hbm, wqr_hbm, wkr_ref,
             wo_hbm, out_ref, c_send, c_recv, wuk_recv, wuv_recv,
             o_mine, o_xnbr, o_peer, o_diag, wq_buf, wqr_buf, wo_buf,
             send_sems, recv_sems, local_sems):
        my_x = lax.axis_index("x")
        my_y = lax.axis_index("y")
        my_z = lax.axis_index("z")
        ypeer = (my_x, 1 - my_y, my_z)
        xnbr = (1 - my_x, my_y, my_z)

        qbase = my_y * HB + my_x * HC
        xn_base = my_y * HB + (1 - my_x) * HC
        pbase = (1 - my_y) * HB + my_x * HC
        dbase = (1 - my_y) * HB + (1 - my_x) * HC

        cp_wq = pltpu.make_async_copy(
            wq_hbm.at[:, pl.ds(qbase, HC)], wq_buf, local_sems.at[0])
        cp_wq.start()
        cp_wqr = pltpu.make_async_copy(
            wqr_hbm.at[:, pl.ds(my_y * (H // 2) * DR + my_x * RC, RC)],
            wqr_buf, local_sems.at[3])
        cp_wqr.start()
        cp_wo0 = pltpu.make_async_copy(
            wo_hbm.at[pl.ds(qbase, HC), :], wo_buf.at[0], local_sems.at[1])
        cp_wo0.start()
        cp_wo1 = pltpu.make_async_copy(
            wo_hbm.at[pl.ds(xn_base, HC), :], wo_buf.at[1], local_sems.at[2])
        cp_wo1.start()

        barrier = pltpu.get_barrier_semaphore()
        for nbr in (ypeer, xnbr):
            pl.semaphore_signal(barrier, inc=1, device_id=nbr,
                                device_id_type=pl.DeviceIdType.MESH)
        pl.semaphore_wait(barrier, 2)

        def rdma(src, dst, i, dev):
            return pltpu.make_async_remote_copy(
                src_ref=src, dst_ref=dst,
                send_sem=send_sems.at[i], recv_sem=recv_sems.at[i],
                device_id=dev, device_id_type=pl.DeviceIdType.MESH)

        rdma_wuk = rdma(wuk_ref.at[:, pl.ds(pbase, HC)], wuk_recv, 0, ypeer)
        rdma_wuv = rdma(wuv_ref.at[:, pl.ds(pbase, HC)], wuv_recv, 1, ypeer)
        rdma_wuk.start()
        rdma_wuv.start()

        xv = x_ref[...].astype(BF16)
        c = jnp.dot(xv, wdkv_ref[...].astype(BF16),
                    preferred_element_type=F32).astype(BF16)
        c_send[...] = c
        rdma_c = rdma(c_send, c_recv, 2, ypeer)
        rdma_c.start()

        cp_wq.wait()
        q = jnp.dot(xv, wq_buf[...].astype(BF16),
                    preferred_element_type=F32).astype(BF16)
        cp_wqr.wait()
        qr = jnp.dot(xv, wqr_buf[...].astype(BF16),
                     preferred_element_type=F32).astype(BF16)
        kr = jnp.dot(xv, wkr_ref[...].astype(BF16),
                     preferred_element_type=F32).astype(BF16)
        qs = (q.astype(F32) * SCALE).astype(BF16)
        qrs = (qr.astype(F32) * SCALE).astype(BF16)

        rdma_wuk.wait()
        rdma_wuv.wait()
        rdma_c.wait()

        cp = c_recv[...]
        k = (jnp.dot(c, wuk_ref[:, pl.ds(qbase, HC)].astype(BF16),
                     preferred_element_type=F32)
             + jnp.dot(cp, wuk_recv[...].astype(BF16),
                       preferred_element_type=F32)).astype(BF16)
        v = (jnp.dot(c, wuv_ref[:, pl.ds(qbase, HC)].astype(BF16),
                     preferred_element_type=F32)
             + jnp.dot(cp, wuv_recv[...].astype(BF16),
                       preferred_element_type=F32)).astype(BF16)

        ox_rdmas, op_rdmas = [], []
        for h in range(H // 4):
            qcat = jnp.concatenate(
                [qs[:, h * DH:(h + 1) * DH], qrs[:, h * DR:(h + 1) * DR]],
                axis=1)
            kcat = jnp.concatenate([k[:, h * DH:(h + 1) * DH], kr], axis=1)
            v_h = v[:, h * DH:(h + 1) * DH]
            p = jnp.exp(lax.dot_general(qcat, kcat, (((1,), (1,)), ((), ())),
                                        preferred_element_type=F32))
            rs = jnp.sum(p, axis=1, keepdims=True)
            o_h = jnp.dot(p.astype(BF16), v_h, preferred_element_type=F32)
            sl = pl.ds(h * DH, DH)
            o_mine[:, sl] = (o_h / rs).astype(BF16)
            rx = rdma(o_mine.at[:, sl], o_xnbr.at[:, sl], 7 + h, xnbr)
            rx.start()
            ox_rdmas.append(rx)
            ry = rdma(o_mine.at[:, sl], o_peer.at[:, sl], 3 + h, ypeer)
            ry.start()
            op_rdmas.append(ry)

        cp_wo0.wait()
        acc = jnp.dot(o_mine[...], wo_buf[0].astype(BF16),
                      preferred_element_type=F32)
        cp_wo2 = pltpu.make_async_copy(
            wo_hbm.at[pl.ds(pbase, HC), :], wo_buf.at[0], local_sems.at[1])
        cp_wo2.start()

        of_rdmas = []
        for h in range(H // 4):
            ox_rdmas[h].wait()
            sl = pl.ds(h * DH, DH)
            rf = rdma(o_xnbr.at[:, sl], o_diag.at[:, sl], 11 + h, ypeer)
            rf.start()
            of_rdmas.append(rf)
        cp_wo1.wait()
        acc = acc + jnp.dot(o_xnbr[...], wo_buf[1].astype(BF16),
                            preferred_element_type=F32)
        cp_wo3 = pltpu.make_async_copy(
            wo_hbm.at[pl.ds(dbase, HC), :], wo_buf.at[1], local_sems.at[2])
        cp_wo3.start()

        for r in op_rdmas:
            r.wait()
        cp_wo2.wait()
        acc = acc + jnp.dot(o_peer[...], wo_buf[0].astype(BF16),
                            preferred_element_type=F32)
        for r in of_rdmas:
            r.wait()
        cp_wo3.wait()
        out_ref[...] = acc + jnp.dot(o_diag[...], wo_buf[1].astype(BF16),
                                     preferred_element_type=F32)

    in_specs = [pl.BlockSpec(memory_space=pltpu.VMEM)] * 8
    in_specs[4] = pl.BlockSpec(memory_space=pl.ANY)
    in_specs[5] = pl.BlockSpec(memory_space=pl.ANY)
    in_specs[7] = pl.BlockSpec(memory_space=pl.ANY)

    out = pl.pallas_call(
        body,
        out_shape=jax.ShapeDtypeStruct((S, D), F32),
        in_specs=in_specs,
        out_specs=pl.BlockSpec(memory_space=pltpu.VMEM),
        scratch_shapes=[
            pltpu.VMEM((S, DC), BF16),
            pltpu.VMEM((S, DC), BF16),
            pltpu.VMEM((DC, HC), F32),
            pltpu.VMEM((DC, HC), F32),
            pltpu.VMEM((S, HC), BF16),
            pltpu.VMEM((S, HC), BF16),
            pltpu.VMEM((S, HC), BF16),
            pltpu.VMEM((S, HC), BF16),
            pltpu.VMEM((D, HC), F32),
            pltpu.VMEM((D, RC), F32),
            pltpu.VMEM((2, HC, D), F32),
            pltpu.SemaphoreType.DMA((15,)),
            pltpu.SemaphoreType.DMA((15,)),
            pltpu.SemaphoreType.DMA((4,)),
        ],
        compiler_params=pltpu.CompilerParams(
            collective_id=0, vmem_limit_bytes=63 * 1024 * 1024),
    )(x.reshape(S, D), Wdkv, Wuk, Wuv, Wq, Wqr, Wkr, Wo)
    return out.reshape(1, S, D)
